# baseline (speedup 1.0000x reference)
import functools

import jax
import jax.numpy as jnp
from jax import lax
from jax.experimental import pallas as pl
from jax.experimental.pallas import tpu as pltpu
from jax.experimental.pallas import tpu_sc as plsc

DIM = 128
BATCH = 16384
NUM_WORKERS = 32
ROWS_PER_WORKER = BATCH // NUM_WORKERS
CHUNK = 64
NUM_CHUNKS = ROWS_PER_WORKER // CHUNK
NUM_PAIRS = NUM_CHUNKS // 2
LANES = 16
GROUPS = DIM // LANES


def _permute_body(x_hbm, idx_hbm, out_hbm, idx_v, in_bufs, out_bufs, sems_in,
                  sems_out):
    wid = lax.axis_index("s") * 2 + lax.axis_index("c")
    base = wid * ROWS_PER_WORKER

    def in_slice(c):
        return x_hbm.at[pl.ds((base + c * CHUNK) * DIM, CHUNK * DIM)]

    def out_slice(c):
        return out_hbm.at[pl.ds((base + c * CHUNK) * DIM, CHUNK * DIM)]

    def start_in(c, s):
        pltpu.make_async_copy(in_slice(c), in_bufs[s], sems_in[s]).start()

    def wait_in(c, s):
        pltpu.make_async_copy(in_slice(c), in_bufs[s], sems_in[s]).wait()

    def start_out(c, s):
        pltpu.make_async_copy(out_bufs[s], out_slice(c), sems_out[s]).start()

    def wait_out(c, s):
        pltpu.make_async_copy(out_bufs[s], out_slice(c), sems_out[s]).wait()

    def compute(s):
        in_v = in_bufs[s]
        out_v = out_bufs[s]

        @plsc.parallel_loop(0, CHUNK, step=1, unroll=2)
        def _row(b):
            rb = b * DIM
            for g in range(GROUPS):
                out_v[pl.ds(rb + g * LANES, LANES)] = plsc.load_gather(
                    in_v, [col_idx[g] + rb]
                )

    start_in(0, 0)
    pltpu.sync_copy(idx_hbm, idx_v)
    col_idx = [idx_v[pl.ds(g * LANES, LANES)] for g in range(GROUPS)]

    def pair_body(p, carry):
        for s in (0, 1):
            c = 2 * p + s

            @pl.when(jnp.logical_or(p < NUM_PAIRS - 1, s == 0))
            def _():
                start_in(c + 1, 1 - s)

            wait_in(c, s)

            @pl.when(p > 0)
            def _():
                wait_out(c - 2, s)

            compute(s)
            start_out(c, s)
        return carry

    lax.fori_loop(0, NUM_PAIRS, pair_body, 0)
    wait_out(NUM_CHUNKS - 2, 0)
    wait_out(NUM_CHUNKS - 1, 1)


_permute = functools.partial(
    pl.kernel,
    mesh=plsc.VectorSubcoreMesh(core_axis_name="c", subcore_axis_name="s"),
    out_type=jax.ShapeDtypeStruct((BATCH * DIM,), jnp.float32),
    scratch_types=[
        pltpu.VMEM((DIM,), jnp.int32),
        [pltpu.VMEM((CHUNK * DIM,), jnp.float32) for _ in range(2)],
        [pltpu.VMEM((CHUNK * DIM,), jnp.float32) for _ in range(2)],
        [pltpu.SemaphoreType.DMA for _ in range(2)],
        [pltpu.SemaphoreType.DMA for _ in range(2)],
    ],
    compiler_params=pltpu.CompilerParams(needs_layout_passes=False),
)(_permute_body)


@jax.jit
def kernel(x, index):
    z = _permute(x.reshape(BATCH * DIM), index).reshape(BATCH, DIM)
    log_det = jnp.zeros((x.shape[0],), dtype=jnp.float32)
    return (z, log_det)


# speedup vs baseline: 1.0473x; 1.0473x over previous
import functools

import jax
import jax.numpy as jnp
from jax import lax
from jax.experimental import pallas as pl
from jax.experimental.pallas import tpu as pltpu
from jax.experimental.pallas import tpu_sc as plsc

DIM = 128
BATCH = 16384
NUM_WORKERS = 32
ROWS_PER_WORKER = BATCH // NUM_WORKERS
CHUNK = 128
NUM_CHUNKS = ROWS_PER_WORKER // CHUNK
NUM_PAIRS = NUM_CHUNKS // 2
LANES = 16
GROUPS = DIM // LANES


def _permute_body(x_hbm, idx_hbm, out_hbm, idx_v, in_bufs, out_bufs, sems_in,
                  sems_out):
    wid = lax.axis_index("s") * 2 + lax.axis_index("c")
    base = wid * ROWS_PER_WORKER

    def in_slice(c):
        return x_hbm.at[pl.ds((base + c * CHUNK) * DIM, CHUNK * DIM)]

    def out_slice(c):
        return out_hbm.at[pl.ds((base + c * CHUNK) * DIM, CHUNK * DIM)]

    def start_in(c, s):
        pltpu.make_async_copy(in_slice(c), in_bufs[s], sems_in[s]).start()

    def wait_in(c, s):
        pltpu.make_async_copy(in_slice(c), in_bufs[s], sems_in[s]).wait()

    def start_out(c, s):
        pltpu.make_async_copy(out_bufs[s], out_slice(c), sems_out[s]).start()

    def wait_out(c, s):
        pltpu.make_async_copy(out_bufs[s], out_slice(c), sems_out[s]).wait()

    def compute(s):
        in_v = in_bufs[s]
        out_v = out_bufs[s]

        @plsc.parallel_loop(0, CHUNK, step=1, unroll=2)
        def _row(b):
            rb = b * DIM
            for g in range(GROUPS):
                out_v[pl.ds(rb + g * LANES, LANES)] = plsc.load_gather(
                    in_v, [col_idx[g] + rb]
                )

    start_in(0, 0)
    pltpu.sync_copy(idx_hbm, idx_v)
    col_idx = [idx_v[pl.ds(g * LANES, LANES)] for g in range(GROUPS)]

    def pair_body(p, carry):
        for s in (0, 1):
            c = 2 * p + s

            @pl.when(jnp.logical_or(p < NUM_PAIRS - 1, s == 0))
            def _():
                start_in(c + 1, 1 - s)

            wait_in(c, s)

            @pl.when(p > 0)
            def _():
                wait_out(c - 2, s)

            compute(s)
            start_out(c, s)
        return carry

    lax.fori_loop(0, NUM_PAIRS, pair_body, 0)
    wait_out(NUM_CHUNKS - 2, 0)
    wait_out(NUM_CHUNKS - 1, 1)


_permute = functools.partial(
    pl.kernel,
    mesh=plsc.VectorSubcoreMesh(core_axis_name="c", subcore_axis_name="s"),
    out_type=jax.ShapeDtypeStruct((BATCH * DIM,), jnp.float32),
    scratch_types=[
        pltpu.VMEM((DIM,), jnp.int32),
        [pltpu.VMEM((CHUNK * DIM,), jnp.float32) for _ in range(2)],
        [pltpu.VMEM((CHUNK * DIM,), jnp.float32) for _ in range(2)],
        [pltpu.SemaphoreType.DMA for _ in range(2)],
        [pltpu.SemaphoreType.DMA for _ in range(2)],
    ],
    compiler_params=pltpu.CompilerParams(needs_layout_passes=False),
)(_permute_body)


@jax.jit
def kernel(x, index):
    z = _permute(x.reshape(BATCH * DIM), index).reshape(BATCH, DIM)
    log_det = jnp.zeros((x.shape[0],), dtype=jnp.float32)
    return (z, log_det)


# speedup vs baseline: 1.0492x; 1.0018x over previous
import functools

import jax
import jax.numpy as jnp
from jax import lax
from jax.experimental import pallas as pl
from jax.experimental.pallas import tpu as pltpu
from jax.experimental.pallas import tpu_sc as plsc

DIM = 128
BATCH = 16384
NUM_WORKERS = 32
ROWS_PER_WORKER = BATCH // NUM_WORKERS
CHUNK = 128
NUM_CHUNKS = ROWS_PER_WORKER // CHUNK
NUM_PAIRS = NUM_CHUNKS // 2
LANES = 16
GROUPS = DIM // LANES


def _permute_body(x_hbm, idx_hbm, out_hbm, idx_v, in_bufs, out_bufs, sems_in,
                  sems_out):
    wid = lax.axis_index("s") * 2 + lax.axis_index("c")
    base = wid * ROWS_PER_WORKER

    def in_slice(c):
        return x_hbm.at[pl.ds((base + c * CHUNK) * DIM, CHUNK * DIM)]

    def out_slice(c):
        return out_hbm.at[pl.ds((base + c * CHUNK) * DIM, CHUNK * DIM)]

    def start_in(c, s):
        pltpu.make_async_copy(in_slice(c), in_bufs[s], sems_in[s]).start()

    def wait_in(c, s):
        pltpu.make_async_copy(in_slice(c), in_bufs[s], sems_in[s]).wait()

    def start_out(c, s):
        pltpu.make_async_copy(out_bufs[s], out_slice(c), sems_out[s]).start()

    def wait_out(c, s):
        pltpu.make_async_copy(out_bufs[s], out_slice(c), sems_out[s]).wait()

    def compute(s):
        in_v = in_bufs[s]
        out_v = out_bufs[s]

        @plsc.parallel_loop(0, CHUNK, step=1, unroll=1)
        def _row(b):
            rb = b * DIM
            for g in range(GROUPS):
                out_v[pl.ds(rb + g * LANES, LANES)] = plsc.load_gather(
                    in_v, [col_idx[g] + rb]
                )

    start_in(0, 0)
    pltpu.sync_copy(idx_hbm, idx_v)
    col_idx = [idx_v[pl.ds(g * LANES, LANES)] for g in range(GROUPS)]

    def pair_body(p, carry):
        for s in (0, 1):
            c = 2 * p + s

            @pl.when(jnp.logical_or(p < NUM_PAIRS - 1, s == 0))
            def _():
                start_in(c + 1, 1 - s)

            wait_in(c, s)

            @pl.when(p > 0)
            def _():
                wait_out(c - 2, s)

            compute(s)
            start_out(c, s)
        return carry

    lax.fori_loop(0, NUM_PAIRS, pair_body, 0)
    wait_out(NUM_CHUNKS - 2, 0)
    wait_out(NUM_CHUNKS - 1, 1)


_permute = functools.partial(
    pl.kernel,
    mesh=plsc.VectorSubcoreMesh(core_axis_name="c", subcore_axis_name="s"),
    out_type=jax.ShapeDtypeStruct((BATCH * DIM,), jnp.float32),
    scratch_types=[
        pltpu.VMEM((DIM,), jnp.int32),
        [pltpu.VMEM((CHUNK * DIM,), jnp.float32) for _ in range(2)],
        [pltpu.VMEM((CHUNK * DIM,), jnp.float32) for _ in range(2)],
        [pltpu.SemaphoreType.DMA for _ in range(2)],
        [pltpu.SemaphoreType.DMA for _ in range(2)],
    ],
    compiler_params=pltpu.CompilerParams(needs_layout_passes=False),
)(_permute_body)


@jax.jit
def kernel(x, index):
    z = _permute(x.reshape(BATCH * DIM), index).reshape(BATCH, DIM)
    log_det = jnp.zeros((x.shape[0],), dtype=jnp.float32)
    return (z, log_det)


# speedup vs baseline: 1.0686x; 1.0185x over previous
import functools

import jax
import jax.numpy as jnp
from jax import lax
from jax.experimental import pallas as pl
from jax.experimental.pallas import tpu as pltpu
from jax.experimental.pallas import tpu_sc as plsc

DIM = 128
BATCH = 16384
NUM_WORKERS = 32
ROWS_PER_WORKER = BATCH // NUM_WORKERS
CHUNK = 128
NUM_CHUNKS = ROWS_PER_WORKER // CHUNK
NUM_PAIRS = NUM_CHUNKS // 2
LANES = 16
GROUPS = DIM // LANES


def _permute_body(x_hbm, idx_hbm, out_hbm, ld_hbm, idx_v, zeros_v, in_bufs,
                  out_bufs, sems_in, sems_out, sem_ld):
    wid = lax.axis_index("s") * 2 + lax.axis_index("c")
    base = wid * ROWS_PER_WORKER

    def in_slice(c):
        return x_hbm.at[pl.ds((base + c * CHUNK) * DIM, CHUNK * DIM)]

    def out_slice(c):
        return out_hbm.at[pl.ds((base + c * CHUNK) * DIM, CHUNK * DIM)]

    def start_in(c, s):
        pltpu.make_async_copy(in_slice(c), in_bufs[s], sems_in[s]).start()

    def wait_in(c, s):
        pltpu.make_async_copy(in_slice(c), in_bufs[s], sems_in[s]).wait()

    def start_out(c, s):
        pltpu.make_async_copy(out_bufs[s], out_slice(c), sems_out[s]).start()

    def wait_out(c, s):
        pltpu.make_async_copy(out_bufs[s], out_slice(c), sems_out[s]).wait()

    def compute(s):
        in_v = in_bufs[s]
        out_v = out_bufs[s]

        @plsc.parallel_loop(0, CHUNK, step=1, unroll=1)
        def _row(b):
            rb = b * DIM
            for g in range(GROUPS):
                out_v[pl.ds(rb + g * LANES, LANES)] = plsc.load_gather(
                    in_v, [col_idx[g] + rb]
                )

    start_in(0, 0)
    pltpu.sync_copy(idx_hbm, idx_v)
    col_idx = [idx_v[pl.ds(g * LANES, LANES)] for g in range(GROUPS)]

    @plsc.parallel_loop(0, ROWS_PER_WORKER, step=LANES)
    def _zrow(i):
        zeros_v[pl.ds(i, LANES)] = jnp.zeros((LANES,), jnp.float32)

    ld_copy = pltpu.async_copy(
        zeros_v, ld_hbm.at[pl.ds(base, ROWS_PER_WORKER)], sem_ld
    )

    def pair_body(p, carry):
        for s in (0, 1):
            c = 2 * p + s

            @pl.when(jnp.logical_or(p < NUM_PAIRS - 1, s == 0))
            def _():
                start_in(c + 1, 1 - s)

            wait_in(c, s)

            @pl.when(p > 0)
            def _():
                wait_out(c - 2, s)

            compute(s)
            start_out(c, s)
        return carry

    lax.fori_loop(0, NUM_PAIRS, pair_body, 0)
    wait_out(NUM_CHUNKS - 2, 0)
    wait_out(NUM_CHUNKS - 1, 1)
    ld_copy.wait()


_permute = functools.partial(
    pl.kernel,
    mesh=plsc.VectorSubcoreMesh(core_axis_name="c", subcore_axis_name="s"),
    out_type=(
        jax.ShapeDtypeStruct((BATCH * DIM,), jnp.float32),
        jax.ShapeDtypeStruct((BATCH,), jnp.float32),
    ),
    scratch_types=[
        pltpu.VMEM((DIM,), jnp.int32),
        pltpu.VMEM((ROWS_PER_WORKER,), jnp.float32),
        [pltpu.VMEM((CHUNK * DIM,), jnp.float32) for _ in range(2)],
        [pltpu.VMEM((CHUNK * DIM,), jnp.float32) for _ in range(2)],
        [pltpu.SemaphoreType.DMA for _ in range(2)],
        [pltpu.SemaphoreType.DMA for _ in range(2)],
        pltpu.SemaphoreType.DMA,
    ],
    compiler_params=pltpu.CompilerParams(needs_layout_passes=False),
)(_permute_body)


@jax.jit
def kernel(x, index):
    z, log_det = _permute(x.reshape(BATCH * DIM), index)
    return (z.reshape(BATCH, DIM), log_det)
